# Initial kernel scaffold; baseline (speedup 1.0000x reference)
import jax
import jax.numpy as jnp
from jax.experimental import pallas as pl


def kernel(mask, edge_index, assign_edge):
    raise NotImplementedError("write your pallas kernel here")


# speedup vs baseline: 388.8965x; 388.8965x over previous
import functools

import jax
import jax.numpy as jnp
from jax import lax
from jax.experimental import pallas as pl
from jax.experimental.pallas import tpu as pltpu
from jax.experimental.pallas import tpu_sc as plsc

_N = 100000
_E = 6400000
_ROWS = 784
_NPAD = _ROWS * 128
_NC = 2
_NS = 16
_NW = _NC * _NS
_EPW = _E // _NW
_K = 4000
_STEPS = _EPW // _K
_SL = _NPAD // _NS
_GAMMA = 0.5

_mesh = plsc.VectorSubcoreMesh(core_axis_name="c", subcore_axis_name="s")


@functools.partial(
    pl.kernel,
    mesh=_mesh,
    compiler_params=pltpu.CompilerParams(needs_layout_passes=False),
    out_type=[
        jax.ShapeDtypeStruct((_NC, _NPAD), jnp.float32),
        jax.ShapeDtypeStruct((_NC, _NPAD), jnp.float32),
    ],
    scratch_types=[
        pltpu.VMEM((_NPAD,), jnp.float32),
        pltpu.VMEM((_K,), jnp.int32),
        pltpu.VMEM((_K,), jnp.int32),
        pltpu.VMEM((_K,), jnp.float32),
        pltpu.VMEM((_K,), jnp.float32),
        pltpu.VMEM_SHARED((_NPAD,), jnp.float32),
        pltpu.VMEM_SHARED((_NPAD,), jnp.float32),
    ],
)
def _edge_pass(ei_hbm, mask_hbm, t_hbm, c_hbm,
               mask_v, i0_v, i1_v, val_v, ones_v, t_sh, c_sh):
    cid = lax.axis_index("c")
    sid = lax.axis_index("s")
    wid = sid * _NC + cid

    pltpu.sync_copy(mask_hbm, mask_v)

    zeros16 = jnp.zeros((16,), jnp.float32)
    ones16 = jnp.ones((16,), jnp.float32)

    def _fill_z(i, carry):
        val_v[pl.ds(pl.multiple_of(i * 16, 16), 16)] = zeros16
        return carry

    lax.fori_loop(0, _K // 16, _fill_z, 0)

    def _fill_o(i, carry):
        ones_v[pl.ds(pl.multiple_of(i * 16, 16), 16)] = ones16
        return carry

    lax.fori_loop(0, _K // 16, _fill_o, 0)

    off = pl.multiple_of(sid * _SL, 8)
    _HSL = _SL // 2
    for h in range(2):
        pltpu.sync_copy(val_v.at[pl.ds(0, _HSL)], t_sh.at[pl.ds(off + h * _HSL, _HSL)])
        pltpu.sync_copy(val_v.at[pl.ds(0, _HSL)], c_sh.at[pl.ds(off + h * _HSL, _HSL)])
    plsc.subcore_barrier()

    ebase = wid * _EPW

    def _step(s, carry):
        base = pl.multiple_of(ebase + s * _K, 8)
        pltpu.sync_copy(ei_hbm.at[pl.ds(base, _K)], i0_v)
        pltpu.sync_copy(ei_hbm.at[pl.ds(base + _E, _K)], i1_v)

        def _gather(j, c2):
            o = pl.multiple_of(j * 16, 16)
            idx = i1_v[pl.ds(o, 16)]
            val_v[pl.ds(o, 16)] = plsc.load_gather(mask_v, [idx])
            return c2

        lax.fori_loop(0, _K // 16, _gather, 0)

        pltpu.sync_copy(val_v, t_sh.at[i0_v], add=True)
        pltpu.sync_copy(ones_v, c_sh.at[i0_v], add=True)
        return carry

    lax.fori_loop(0, _STEPS, _step, 0)

    plsc.subcore_barrier()
    pltpu.sync_copy(t_sh.at[pl.ds(off, _SL)], t_hbm.at[cid, pl.ds(off, _SL)])
    pltpu.sync_copy(c_sh.at[pl.ds(off, _SL)], c_hbm.at[cid, pl.ds(off, _SL)])


def _fin_body(m_ref, t_ref, c_ref, o_ref):
    m = m_ref[...]
    t = t_ref[...]
    c = c_ref[...]
    ts = t[0] + t[1]
    cs = c[0] + c[1]
    o_ref[...] = (1.0 - _GAMMA) * m + (_GAMMA * 0.5) * (cs * m + ts) / jnp.maximum(cs, 1.0)


_finalize = functools.partial(
    pl.pallas_call,
    _fin_body,
    out_shape=jax.ShapeDtypeStruct((_ROWS, 128), jnp.float32),
)()


def kernel(mask, edge_index, assign_edge):
    del assign_edge
    mask_pad = jnp.pad(mask.reshape(-1), (0, _NPAD - _N))
    t, c = _edge_pass(edge_index.reshape(-1), mask_pad)
    out = _finalize(
        mask_pad.reshape(_ROWS, 128),
        t.reshape(_NC, _ROWS, 128),
        c.reshape(_NC, _ROWS, 128),
    )
    return out.reshape(-1)[:_N].reshape(_N, 1)


# speedup vs baseline: 604.6782x; 1.5549x over previous
import functools

import jax
import jax.numpy as jnp
from jax import lax
from jax.experimental import pallas as pl
from jax.experimental.pallas import tpu as pltpu
from jax.experimental.pallas import tpu_sc as plsc

_N = 100000
_E = 6400000
_ROWS = 784
_NPAD = _ROWS * 128
_NC = 2
_NS = 16
_NW = _NC * _NS
_EPW = _E // _NW
_K = 2000
_STEPS = _EPW // _K
_SL = _NPAD // _NS
_GAMMA = 0.5

_mesh = plsc.VectorSubcoreMesh(core_axis_name="c", subcore_axis_name="s")


@functools.partial(
    pl.kernel,
    mesh=_mesh,
    compiler_params=pltpu.CompilerParams(
        needs_layout_passes=False, use_tc_tiling_on_sc=False
    ),
    out_type=[
        jax.ShapeDtypeStruct((_NC, _NPAD), jnp.float32),
        jax.ShapeDtypeStruct((_NC, _NPAD), jnp.float32),
    ],
    scratch_types=[
        pltpu.VMEM((_NPAD,), jnp.float32),
        pltpu.VMEM((2, _K), jnp.int32),
        pltpu.VMEM((2, _K), jnp.int32),
        pltpu.VMEM((2, _K), jnp.float32),
        pltpu.VMEM((_K,), jnp.float32),
        pltpu.VMEM_SHARED((_NPAD,), jnp.float32),
        pltpu.VMEM_SHARED((_NPAD,), jnp.float32),
        pltpu.SemaphoreType.DMA,
        pltpu.SemaphoreType.DMA,
        pltpu.SemaphoreType.DMA,
        pltpu.SemaphoreType.DMA,
        pltpu.SemaphoreType.DMA,
        pltpu.SemaphoreType.DMA,
        pltpu.SemaphoreType.DMA,
        pltpu.SemaphoreType.DMA,
    ],
)
def _edge_pass(ei_hbm, mask_hbm, t_hbm, c_hbm,
               mask_v, i0_v, i1_v, val_v, ones_v, t_sh, c_sh,
               sl0_a, sl0_b, sl1_a, sl1_b, st_a, st_b, sc_a, sc_b):
    cid = lax.axis_index("c")
    sid = lax.axis_index("s")
    wid = sid * _NC + cid
    sl0 = (sl0_a, sl0_b)
    sl1 = (sl1_a, sl1_b)
    st = (st_a, st_b)
    sc = (sc_a, sc_b)

    pltpu.sync_copy(mask_hbm, mask_v)

    zeros16 = jnp.zeros((16,), jnp.float32)
    ones16 = jnp.ones((16,), jnp.float32)

    def _fill_z(i, carry):
        o = pl.multiple_of(i * 16, 16)
        val_v[0, pl.ds(o, 16)] = zeros16
        ones_v[pl.ds(o, 16)] = ones16
        return carry

    lax.fori_loop(0, _K // 16, _fill_z, 0)

    off = pl.multiple_of(sid * _SL, 8)
    pos = 0
    while pos < _SL:
        n = min(_K, _SL - pos)
        pltpu.sync_copy(val_v.at[0, pl.ds(0, n)], t_sh.at[pl.ds(off + pos, n)])
        pltpu.sync_copy(val_v.at[0, pl.ds(0, n)], c_sh.at[pl.ds(off + pos, n)])
        pos += n
    plsc.subcore_barrier()

    ebase = wid * _EPW

    def _load(s, p):
        base = pl.multiple_of(ebase + s * _K, 8)
        pltpu.async_copy(ei_hbm.at[pl.ds(base, _K)], i0_v.at[p], sl0[p])
        pltpu.async_copy(ei_hbm.at[pl.ds(base + _E, _K)], i1_v.at[p], sl1[p])

    def _wait_load(s, p):
        base = pl.multiple_of(ebase + s * _K, 8)
        pltpu.make_async_copy(ei_hbm.at[pl.ds(base, _K)], i0_v.at[p], sl0[p]).wait()
        pltpu.make_async_copy(ei_hbm.at[pl.ds(base + _E, _K)], i1_v.at[p], sl1[p]).wait()

    def _gather(p):
        def _g(j, c2):
            o = pl.multiple_of(j * 16, 16)
            idx = i1_v[p, pl.ds(o, 16)]
            val_v[p, pl.ds(o, 16)] = plsc.load_gather(mask_v, [idx])
            return c2

        lax.fori_loop(0, _K // 16, _g, 0)

    def _scatter(p):
        pltpu.async_copy(val_v.at[p], t_sh.at[i0_v.at[p]], st[p], add=True)
        pltpu.async_copy(ones_v, c_sh.at[i0_v.at[p]], sc[p], add=True)

    def _wait_scatter(p):
        pltpu.make_async_copy(val_v.at[p], t_sh.at[i0_v.at[p]], st[p]).wait()
        pltpu.make_async_copy(ones_v, c_sh.at[i0_v.at[p]], sc[p]).wait()

    _load(0, 0)

    def _iter(g, carry):
        s0 = g * 2
        _wait_load(s0, 0)
        _gather(0)

        @pl.when(g > 0)
        def _():
            _wait_scatter(1)

        _load(s0 + 1, 1)
        _scatter(0)
        _wait_load(s0 + 1, 1)
        _gather(1)
        _wait_scatter(0)

        @pl.when(g + 1 < _STEPS // 2)
        def _():
            _load(s0 + 2, 0)

        _scatter(1)
        return carry

    lax.fori_loop(0, _STEPS // 2, _iter, 0)
    _wait_scatter(1)

    plsc.subcore_barrier()
    pltpu.sync_copy(t_sh.at[pl.ds(off, _SL)], t_hbm.at[cid, pl.ds(off, _SL)])
    pltpu.sync_copy(c_sh.at[pl.ds(off, _SL)], c_hbm.at[cid, pl.ds(off, _SL)])


def _fin_body(m_ref, t_ref, c_ref, o_ref):
    m = m_ref[...]
    t = t_ref[...]
    c = c_ref[...]
    ts = t[0] + t[1]
    cs = c[0] + c[1]
    o_ref[...] = (1.0 - _GAMMA) * m + (_GAMMA * 0.5) * (cs * m + ts) / jnp.maximum(cs, 1.0)


_finalize = pl.pallas_call(
    _fin_body,
    out_shape=jax.ShapeDtypeStruct((_ROWS, 128), jnp.float32),
)


def kernel(mask, edge_index, assign_edge):
    del assign_edge
    mask_pad = jnp.pad(mask.reshape(-1), (0, _NPAD - _N))
    t, c = _edge_pass(edge_index.reshape(-1), mask_pad)
    out = _finalize(
        mask_pad.reshape(_ROWS, 128),
        t.reshape(_NC, _ROWS, 128),
        c.reshape(_NC, _ROWS, 128),
    )
    return out.reshape(-1)[:_N].reshape(_N, 1)


# speedup vs baseline: 670.3758x; 1.1086x over previous
import functools

import jax
import jax.numpy as jnp
from jax import lax
from jax.experimental import pallas as pl
from jax.experimental.pallas import tpu as pltpu
from jax.experimental.pallas import tpu_sc as plsc

_N = 100000
_E = 6400000
_ROWS = 784
_NPAD = _ROWS * 128
_NC = 2
_NS = 16
_NW = _NC * _NS
_EPW = _E // _NW
_K = 2000
_STEPS = _EPW // _K
_SL = _NPAD // _NS
_GAMMA = 0.5

_mesh = plsc.VectorSubcoreMesh(core_axis_name="c", subcore_axis_name="s")


@functools.partial(
    pl.kernel,
    mesh=_mesh,
    compiler_params=pltpu.CompilerParams(
        needs_layout_passes=False, use_tc_tiling_on_sc=False
    ),
    out_type=[
        jax.ShapeDtypeStruct((_NC, _NPAD), jnp.float32),
        jax.ShapeDtypeStruct((_NC, _NPAD), jnp.float32),
    ],
    scratch_types=[
        pltpu.VMEM((_NPAD,), jnp.float32),
        pltpu.VMEM((2, _K), jnp.int32),
        pltpu.VMEM((2, _K), jnp.int32),
        pltpu.VMEM((2, _K), jnp.float32),
        pltpu.VMEM((_K,), jnp.float32),
        pltpu.VMEM_SHARED((_NPAD,), jnp.float32),
        pltpu.VMEM_SHARED((_NPAD,), jnp.float32),
        pltpu.SemaphoreType.DMA,
        pltpu.SemaphoreType.DMA,
        pltpu.SemaphoreType.DMA,
        pltpu.SemaphoreType.DMA,
        pltpu.SemaphoreType.DMA,
        pltpu.SemaphoreType.DMA,
        pltpu.SemaphoreType.DMA,
        pltpu.SemaphoreType.DMA,
    ],
)
def _edge_pass(ei_hbm, mask_hbm, t_hbm, c_hbm,
               mask_v, i0_v, i1_v, val_v, ones_v, t_sh, c_sh,
               sl0_a, sl0_b, sl1_a, sl1_b, st_a, st_b, sc_a, sc_b):
    cid = lax.axis_index("c")
    sid = lax.axis_index("s")
    wid = sid * _NC + cid
    sl0 = (sl0_a, sl0_b)
    sl1 = (sl1_a, sl1_b)
    st = (st_a, st_b)
    sc = (sc_a, sc_b)

    pltpu.sync_copy(mask_hbm, mask_v)

    zeros16 = jnp.zeros((16,), jnp.float32)
    ones16 = jnp.ones((16,), jnp.float32)

    def _fill_z(i, carry):
        o = pl.multiple_of(i * 16, 16)
        val_v[0, pl.ds(o, 16)] = zeros16
        ones_v[pl.ds(o, 16)] = ones16
        return carry

    lax.fori_loop(0, _K // 16, _fill_z, 0)

    off = pl.multiple_of(sid * _SL, 8)
    pos = 0
    while pos < _SL:
        n = min(_K, _SL - pos)
        pltpu.sync_copy(val_v.at[0, pl.ds(0, n)], t_sh.at[pl.ds(off + pos, n)])
        pltpu.sync_copy(val_v.at[0, pl.ds(0, n)], c_sh.at[pl.ds(off + pos, n)])
        pos += n
    plsc.subcore_barrier()

    ebase = wid * _EPW

    def _load(s, p):
        base = pl.multiple_of(ebase + s * _K, 8)
        pltpu.async_copy(ei_hbm.at[pl.ds(base, _K)], i0_v.at[p], sl0[p])
        pltpu.async_copy(ei_hbm.at[pl.ds(base + _E, _K)], i1_v.at[p], sl1[p])

    def _wait_load(s, p):
        base = pl.multiple_of(ebase + s * _K, 8)
        pltpu.make_async_copy(ei_hbm.at[pl.ds(base, _K)], i0_v.at[p], sl0[p]).wait()
        pltpu.make_async_copy(ei_hbm.at[pl.ds(base + _E, _K)], i1_v.at[p], sl1[p]).wait()

    def _gather(p):
        def _g(j, c2):
            base_o = pl.multiple_of(j * 64, 64)
            for u in range(4):
                o = base_o + u * 16
                idx = i1_v[p, pl.ds(o, 16)]
                val_v[p, pl.ds(o, 16)] = plsc.load_gather(mask_v, [idx])
            return c2

        lax.fori_loop(0, _K // 64, _g, 0)

    def _scatter(p):
        pltpu.async_copy(val_v.at[p], t_sh.at[i0_v.at[p]], st[p], add=True)
        pltpu.async_copy(ones_v, c_sh.at[i0_v.at[p]], sc[p], add=True)

    def _wait_scatter(p):
        pltpu.make_async_copy(val_v.at[p], t_sh.at[i0_v.at[p]], st[p]).wait()
        pltpu.make_async_copy(ones_v, c_sh.at[i0_v.at[p]], sc[p]).wait()

    _load(0, 0)

    def _iter(g, carry):
        s0 = g * 2
        _wait_load(s0, 0)
        _gather(0)

        @pl.when(g > 0)
        def _():
            _wait_scatter(1)

        _load(s0 + 1, 1)
        _scatter(0)
        _wait_load(s0 + 1, 1)
        _gather(1)
        _wait_scatter(0)

        @pl.when(g + 1 < _STEPS // 2)
        def _():
            _load(s0 + 2, 0)

        _scatter(1)
        return carry

    lax.fori_loop(0, _STEPS // 2, _iter, 0)
    _wait_scatter(1)

    plsc.subcore_barrier()
    pltpu.sync_copy(t_sh.at[pl.ds(off, _SL)], t_hbm.at[cid, pl.ds(off, _SL)])
    pltpu.sync_copy(c_sh.at[pl.ds(off, _SL)], c_hbm.at[cid, pl.ds(off, _SL)])


def _fin_body(m_ref, t_ref, c_ref, o_ref):
    m = m_ref[...]
    t = t_ref[...]
    c = c_ref[...]
    ts = t[0] + t[1]
    cs = c[0] + c[1]
    o_ref[...] = (1.0 - _GAMMA) * m + (_GAMMA * 0.5) * (cs * m + ts) / jnp.maximum(cs, 1.0)


_finalize = pl.pallas_call(
    _fin_body,
    out_shape=jax.ShapeDtypeStruct((_ROWS, 128), jnp.float32),
)


def kernel(mask, edge_index, assign_edge):
    del assign_edge
    mask_pad = jnp.pad(mask.reshape(-1), (0, _NPAD - _N))
    t, c = _edge_pass(edge_index.reshape(-1), mask_pad)
    out = _finalize(
        mask_pad.reshape(_ROWS, 128),
        t.reshape(_NC, _ROWS, 128),
        c.reshape(_NC, _ROWS, 128),
    )
    return out.reshape(-1)[:_N].reshape(_N, 1)
